# baseline (speedup 1.0000x reference)
import functools

import jax
import jax.numpy as jnp
from jax import lax
from jax.experimental import pallas as pl
from jax.experimental.pallas import tpu as pltpu
from jax.experimental.pallas import tpu_sc as plsc

_B, _N, _K, _R = 8, 2048, 16, 2
_M = 512
_NBLK = _N // _M
_D = 128
_NW = 32
_TOT = _B * _N * _K
_PW = _TOT // _NW
_CH = 128
_NCH = _PW // _CH


def _topk_cell(xyz_ref, xyzT_ref,
               W1T_ref, b1_ref, W2T_ref, b2_ref,
               idx_ref, tbl_ref, kk_ref):
    b = pl.program_id(0)
    m = pl.program_id(1)
    f32 = jnp.float32
    x3 = xyz_ref[0]
    Q = xyzT_ref[0, pl.ds(m * _M, _M), :]

    P1 = jnp.maximum(jnp.dot(Q, W1T_ref[...], preferred_element_type=f32)
                     + b1_ref[...], 0.0)
    fq = jnp.maximum(jnp.dot(P1, W2T_ref[...], preferred_element_type=f32)
                     + b2_ref[...], 0.0)
    tbl_ref[0] = jnp.concatenate(
        [fq, Q, jnp.zeros((_M, _D - 67), f32)], axis=1)

    sq_all = jnp.sum(x3 * x3, axis=0, keepdims=True)
    sq_q = jnp.sum(Q * Q, axis=1, keepdims=True)
    dotQ = jnp.dot(Q, x3, preferred_element_type=f32)
    lane = lax.broadcasted_iota(jnp.int32, (_M, 128), 1)
    rid = lax.broadcasted_iota(jnp.int32, (_M, 1), 0) + m * _M
    nch = _N // 128

    base = b * _N
    self_col = rid + base
    macc = None
    for j in range(nch):
        sl = pl.ds(j * 128, 128)
        lo, hi = j * 128, (j + 1) * 128
        d2c = jnp.maximum(sq_q + sq_all[:, lo:hi] - 2.0 * dotQ[:, lo:hi], 0.0)
        lj = lane + (j * 128)
        bch = (lax.bitcast_convert_type(d2c, jnp.int32) & jnp.int32(-2048)) | lj
        bch = jnp.where(lj == rid, jnp.int32(0x7FFFFFFF), bch)
        kk_ref[:, sl] = bch
        macc = bch if macc is None else jnp.minimum(macc, bch)
    kmin = jnp.min(macc, axis=1, keepdims=True)
    cols = [self_col, (kmin & 2047) + base]
    for _ in range(14):
        prev = kmin
        macc = None
        for j in range(nch):
            sl = pl.ds(j * 128, 128)
            c = kk_ref[:, sl]
            c = jnp.where(c == prev, jnp.int32(0x7FFFFFFF), c)
            kk_ref[:, sl] = c
            macc = c if macc is None else jnp.minimum(macc, c)
        kmin = jnp.min(macc, axis=1, keepdims=True)
        cols.append((kmin & 2047) + base)
    idx_ref[0] = jnp.concatenate(cols, axis=1)


def _make_sc_gather(pw, nch):
  def _sc_gather(tbl_hbm, idx_hbm, out_hbm, idx_v, buf0, buf1, sem0, sem1):
    wid = lax.axis_index("s") * 2 + lax.axis_index("c")
    base = wid * pw
    pltpu.sync_copy(idx_hbm.at[pl.ds(base, pw)], idx_v)

    def start(st, buf, sem):
        pltpu.async_copy(tbl_hbm.at[idx_v.at[pl.ds(st, _CH)]], buf, sem)

    def wait(buf, sem):
        pltpu.make_async_copy(tbl_hbm.at[pl.ds(0, _CH)], buf, sem).wait()

    def out(st, buf):
        pltpu.sync_copy(buf, out_hbm.at[pl.ds(base + st, _CH)])

    start(0, buf0, sem0)

    def body(i, carry):
        st = 2 * i * _CH
        start(st + _CH, buf1, sem1)
        wait(buf0, sem0)
        out(st, buf0)
        start(st + 2 * _CH, buf0, sem0)
        wait(buf1, sem1)
        out(st + _CH, buf1)
        return carry

    lax.fori_loop(0, nch // 2 - 1, body, 0)
    st = (nch - 2) * _CH
    start(st + _CH, buf1, sem1)
    wait(buf0, sem0)
    out(st, buf0)
    wait(buf1, sem1)
    out(st + _CH, buf1)
  return _sc_gather


def _mlp_cell(g_ref,
              WpT_ref, WsT_ref, bs_ref,
              Wa1T_ref, ba1_ref, Wa2T_ref, ba2_ref,
              Wm1aT_ref, Wm1bT_ref, bm1_ref, Wm2T_ref, bm2_ref,
              out_ref):
    f32 = jnp.float32
    Gb = g_ref[...]
    nb0 = Gb[0, 0]
    fq = nb0[:, 0:64]
    Q = nb0[:, 64:67]

    WpT = WpT_ref[...]
    Wa1T = Wa1T_ref[...]
    ba1 = ba1_ref[...]
    Wa2T = Wa2T_ref[...]
    ba2 = ba2_ref[...]

    lg0 = jnp.dot(jnp.maximum(ba1, 0.0), Wa2T, preferred_element_type=f32) + ba2

    logits = [lg0]
    fks = []
    planes = None
    for k in range(1, _K):
        nb = Gb[k, 0]
        nbf = nb[:, 0:64]
        nbx = nb[:, 64:67]

        fk = nbf - fq
        a = jnp.maximum(jnp.dot(fk, Wa1T, preferred_element_type=f32) + ba1, 0.0)
        lg = jnp.dot(a, Wa2T, preferred_element_type=f32) + ba2
        logits.append(lg)
        fks.append(fk)

        v = nbx - Q
        nrm = jnp.sqrt(jnp.sum(v * v, axis=1, keepdims=True)) + 1e-8
        p = jnp.dot(v, WpT, preferred_element_type=f32)
        c = p * jnp.abs(p) / nrm
        planes = c if planes is None else jnp.maximum(planes, c)

    mx = functools.reduce(jnp.maximum, logits)
    den = jnp.exp(jnp.broadcast_to(logits[0], mx.shape) - mx)
    num = None
    for k in range(1, _K):
        e = jnp.exp(logits[k] - mx)
        den = den + e
        t = e * fks[k - 1]
        num = t if num is None else num + t
    f_att = num / den

    f_shapes = jnp.dot(planes, WsT_ref[...], preferred_element_type=f32) + bs_ref[...]

    h = jnp.maximum(jnp.dot(f_att, Wm1aT_ref[...], preferred_element_type=f32)
                    + jnp.dot(f_shapes, Wm1bT_ref[...], preferred_element_type=f32)
                    + bm1_ref[...], 0.0)
    o = jnp.dot(h, Wm2T_ref[...], preferred_element_type=f32) + bm2_ref[...]

    Qe = jnp.concatenate([Q[:, 0:1], Q[:, 0:1], Q[:, 1:2], Q[:, 1:2],
                          Q[:, 2:3], Q[:, 2:3]], axis=1)
    out_ref[0] = Qe + 0.15 * o


def kernel(xyz, Wp, Ws, bs, W1, b1, W2, b2, Wa1, ba1, Wa2, ba2, Wm1, bm1, Wm2, bm2):
    Bsz, C, Np = xyz.shape
    f32 = jnp.float32
    xyzT = jnp.transpose(xyz, (0, 2, 1))

    row = lambda v: v.reshape(1, -1)
    wsA = [W1.T, row(b1), W2.T, row(b2)]
    wspecsA = [pl.BlockSpec(w.shape, lambda b, m: (0,) * w.ndim) for w in wsA]
    wsC = [Wp.T, Ws.T, row(bs), Wa1.T, row(ba1), Wa2.T, row(ba2),
           Wm1[:, :64].T, Wm1[:, 64:].T, row(bm1), Wm2.T, row(bm2)]
    wspecsC = [pl.BlockSpec(w.shape, lambda b, m: (0,) * w.ndim) for w in wsC]
    mesh = plsc.VectorSubcoreMesh(core_axis_name="c", subcore_axis_name="s")

    _NG = 4
    nb = _B // _NG
    tot = nb * _N * _K
    pw = tot // _NW
    nch = pw // _CH
    sc_gather = _make_sc_gather(pw, nch)

    outs = []
    for g in range(_NG):
        xyz_g = xyz[g * nb:(g + 1) * nb]
        xyzT_g = xyzT[g * nb:(g + 1) * nb]

        idxg, tblA = pl.pallas_call(
            _topk_cell,
            grid=(nb, _NBLK),
            in_specs=[
                pl.BlockSpec((1, 3, _N), lambda b, m: (b, 0, 0)),
                pl.BlockSpec((1, _N, 3), lambda b, m: (b, 0, 0)),
            ] + wspecsA,
            out_specs=[
                pl.BlockSpec((1, _M, _K), lambda b, m: (b, m, 0)),
                pl.BlockSpec((1, _M, _D), lambda b, m: (b, m, 0)),
            ],
            out_shape=[
                jax.ShapeDtypeStruct((nb, _N, _K), jnp.int32),
                jax.ShapeDtypeStruct((nb, _N, _D), f32),
            ],
            scratch_shapes=[pltpu.VMEM((_M, _N), jnp.int32)],
        )(xyz_g, xyzT_g, *wsA)

        tbl = tblA.reshape(nb * _N, _D)
        idx_flat = jnp.transpose(idxg, (2, 0, 1)).reshape(tot)

        G = pl.kernel(
            sc_gather,
            mesh=mesh,
            out_type=jax.ShapeDtypeStruct((tot, _D), f32),
            scratch_types=[
                pltpu.VMEM((pw,), jnp.int32),
                pltpu.VMEM((_CH, _D), f32),
                pltpu.VMEM((_CH, _D), f32),
                pltpu.SemaphoreType.DMA,
                pltpu.SemaphoreType.DMA,
            ],
        )(tbl, idx_flat)

        G4 = G.reshape(_K, nb, _N, _D)
        out_pm = pl.pallas_call(
            _mlp_cell,
            grid=(nb, _NBLK),
            in_specs=[
                pl.BlockSpec((_K, 1, _M, _D), lambda b, m: (0, b, m, 0)),
            ] + wspecsC,
            out_specs=pl.BlockSpec((1, _M, 6), lambda b, m: (b, m, 0)),
            out_shape=jax.ShapeDtypeStruct((nb, _N, 6), f32),
        )(G4, *wsC)
        outs.append(out_pm)

    out_all = jnp.concatenate(outs, axis=0)
    return out_all.transpose(0, 2, 1).reshape(Bsz, 3, _R * Np)


# speedup vs baseline: 1.0329x; 1.0329x over previous
import functools

import jax
import jax.numpy as jnp
from jax import lax
from jax.experimental import pallas as pl
from jax.experimental.pallas import tpu as pltpu
from jax.experimental.pallas import tpu_sc as plsc

_B, _N, _K, _R = 8, 2048, 16, 2
_M = 512
_NBLK = _N // _M
_D = 128
_NW = 32
_TOT = _B * _N * _K
_PW = _TOT // _NW
_CH = 128
_NCH = _PW // _CH


def _topk_cell(xyz_ref, xyzT_ref,
               W1T_ref, b1_ref, W2T_ref, b2_ref,
               idx_ref, tbl_ref, kk_ref):
    b = pl.program_id(0)
    m = pl.program_id(1)
    f32 = jnp.float32
    x3 = xyz_ref[0]
    Q = xyzT_ref[0, pl.ds(m * _M, _M), :]

    P1 = jnp.maximum(jnp.dot(Q, W1T_ref[...], preferred_element_type=f32)
                     + b1_ref[...], 0.0)
    fq = jnp.maximum(jnp.dot(P1, W2T_ref[...], preferred_element_type=f32)
                     + b2_ref[...], 0.0)
    tbl_ref[0] = jnp.concatenate(
        [fq, Q, jnp.zeros((_M, _D - 67), f32)], axis=1)

    sq_all = jnp.sum(x3 * x3, axis=0, keepdims=True)
    sq_q = jnp.sum(Q * Q, axis=1, keepdims=True)
    dotQ = jnp.dot(Q, x3, preferred_element_type=f32)
    lane = lax.broadcasted_iota(jnp.int32, (_M, 128), 1)
    rid = lax.broadcasted_iota(jnp.int32, (_M, 1), 0) + m * _M
    nch = _N // 128

    base = b * _N
    self_col = rid + base
    maxi = jnp.int32(0x7FFFFFFF)

    def glob2(a1, a2):
        m1 = jnp.min(a1, axis=1, keepdims=True)
        m2 = jnp.min(jnp.where(a1 == m1, a2, a1), axis=1, keepdims=True)
        return m1, m2

    a1 = a2 = None
    for j in range(nch):
        sl = pl.ds(j * 128, 128)
        lo, hi = j * 128, (j + 1) * 128
        d2c = jnp.maximum(sq_q + sq_all[:, lo:hi] - 2.0 * dotQ[:, lo:hi], 0.0)
        lj = lane + (j * 128)
        bch = (lax.bitcast_convert_type(d2c, jnp.int32) & jnp.int32(-2048)) | lj
        bch = jnp.where(lj == rid, maxi, bch)
        kk_ref[:, sl] = bch
        if a1 is None:
            a1, a2 = bch, jnp.full_like(bch, maxi)
        else:
            t = jnp.maximum(a1, bch)
            a1 = jnp.minimum(a1, bch)
            a2 = jnp.minimum(a2, t)
    m1, m2 = glob2(a1, a2)
    cols = [self_col, (m1 & 2047) + base, (m2 & 2047) + base]
    for _ in range(6):
        prev2 = m2
        a1 = a2 = None
        for j in range(nch):
            sl = pl.ds(j * 128, 128)
            c = kk_ref[:, sl]
            c = jnp.where(c <= prev2, maxi, c)
            kk_ref[:, sl] = c
            if a1 is None:
                a1, a2 = c, jnp.full_like(c, maxi)
            else:
                t = jnp.maximum(a1, c)
                a1 = jnp.minimum(a1, c)
                a2 = jnp.minimum(a2, t)
        m1, m2 = glob2(a1, a2)
        cols.append((m1 & 2047) + base)
        cols.append((m2 & 2047) + base)
    prev2 = m2
    macc = None
    for j in range(nch):
        lo, hi = j * 128, (j + 1) * 128
        c = kk_ref[:, pl.ds(j * 128, 128)]
        c = jnp.where(c <= prev2, maxi, c)
        macc = c if macc is None else jnp.minimum(macc, c)
    mlast = jnp.min(macc, axis=1, keepdims=True)
    cols.append((mlast & 2047) + base)
    idx_ref[0] = jnp.concatenate(cols, axis=1)


def _make_sc_gather(pw, nch):
  def _sc_gather(tbl_hbm, idx_hbm, out_hbm, idx_v, buf0, buf1, sem0, sem1):
    wid = lax.axis_index("s") * 2 + lax.axis_index("c")
    base = wid * pw
    pltpu.sync_copy(idx_hbm.at[pl.ds(base, pw)], idx_v)

    def start(st, buf, sem):
        pltpu.async_copy(tbl_hbm.at[idx_v.at[pl.ds(st, _CH)]], buf, sem)

    def wait(buf, sem):
        pltpu.make_async_copy(tbl_hbm.at[pl.ds(0, _CH)], buf, sem).wait()

    def out(st, buf):
        pltpu.sync_copy(buf, out_hbm.at[pl.ds(base + st, _CH)])

    start(0, buf0, sem0)

    def body(i, carry):
        st = 2 * i * _CH
        start(st + _CH, buf1, sem1)
        wait(buf0, sem0)
        out(st, buf0)
        start(st + 2 * _CH, buf0, sem0)
        wait(buf1, sem1)
        out(st + _CH, buf1)
        return carry

    lax.fori_loop(0, nch // 2 - 1, body, 0)
    st = (nch - 2) * _CH
    start(st + _CH, buf1, sem1)
    wait(buf0, sem0)
    out(st, buf0)
    wait(buf1, sem1)
    out(st + _CH, buf1)
  return _sc_gather


def _mlp_cell(g_ref,
              WpT_ref, WsT_ref, bs_ref,
              Wa1T_ref, ba1_ref, Wa2T_ref, ba2_ref,
              Wm1aT_ref, Wm1bT_ref, bm1_ref, Wm2T_ref, bm2_ref,
              out_ref):
    f32 = jnp.float32
    Gb = g_ref[...]
    nb0 = Gb[0, 0]
    fq = nb0[:, 0:64]
    Q = nb0[:, 64:67]

    WpT = WpT_ref[...]
    Wa1T = Wa1T_ref[...]
    ba1 = ba1_ref[...]
    Wa2T = Wa2T_ref[...]
    ba2 = ba2_ref[...]

    lg0 = jnp.dot(jnp.maximum(ba1, 0.0), Wa2T, preferred_element_type=f32) + ba2

    logits = [lg0]
    fks = []
    planes = None
    for k in range(1, _K):
        nb = Gb[k, 0]
        nbf = nb[:, 0:64]
        nbx = nb[:, 64:67]

        fk = nbf - fq
        a = jnp.maximum(jnp.dot(fk, Wa1T, preferred_element_type=f32) + ba1, 0.0)
        lg = jnp.dot(a, Wa2T, preferred_element_type=f32) + ba2
        logits.append(lg)
        fks.append(fk)

        v = nbx - Q
        nrm = jnp.sqrt(jnp.sum(v * v, axis=1, keepdims=True)) + 1e-8
        p = jnp.dot(v, WpT, preferred_element_type=f32)
        c = p * jnp.abs(p) / nrm
        planes = c if planes is None else jnp.maximum(planes, c)

    mx = functools.reduce(jnp.maximum, logits)
    den = jnp.exp(jnp.broadcast_to(logits[0], mx.shape) - mx)
    num = None
    for k in range(1, _K):
        e = jnp.exp(logits[k] - mx)
        den = den + e
        t = e * fks[k - 1]
        num = t if num is None else num + t
    f_att = num / den

    f_shapes = jnp.dot(planes, WsT_ref[...], preferred_element_type=f32) + bs_ref[...]

    h = jnp.maximum(jnp.dot(f_att, Wm1aT_ref[...], preferred_element_type=f32)
                    + jnp.dot(f_shapes, Wm1bT_ref[...], preferred_element_type=f32)
                    + bm1_ref[...], 0.0)
    o = jnp.dot(h, Wm2T_ref[...], preferred_element_type=f32) + bm2_ref[...]

    Qe = jnp.concatenate([Q[:, 0:1], Q[:, 0:1], Q[:, 1:2], Q[:, 1:2],
                          Q[:, 2:3], Q[:, 2:3]], axis=1)
    out_ref[0] = Qe + 0.15 * o


def kernel(xyz, Wp, Ws, bs, W1, b1, W2, b2, Wa1, ba1, Wa2, ba2, Wm1, bm1, Wm2, bm2):
    Bsz, C, Np = xyz.shape
    f32 = jnp.float32
    xyzT = jnp.transpose(xyz, (0, 2, 1))

    row = lambda v: v.reshape(1, -1)
    wsA = [W1.T, row(b1), W2.T, row(b2)]
    wspecsA = [pl.BlockSpec(w.shape, lambda b, m: (0,) * w.ndim) for w in wsA]
    wsC = [Wp.T, Ws.T, row(bs), Wa1.T, row(ba1), Wa2.T, row(ba2),
           Wm1[:, :64].T, Wm1[:, 64:].T, row(bm1), Wm2.T, row(bm2)]
    wspecsC = [pl.BlockSpec(w.shape, lambda b, m: (0,) * w.ndim) for w in wsC]
    mesh = plsc.VectorSubcoreMesh(core_axis_name="c", subcore_axis_name="s")

    _NG = 2
    nb = _B // _NG
    tot = nb * _N * _K
    pw = tot // _NW
    nch = pw // _CH
    sc_gather = _make_sc_gather(pw, nch)

    outs = []
    for g in range(_NG):
        xyz_g = xyz[g * nb:(g + 1) * nb]
        xyzT_g = xyzT[g * nb:(g + 1) * nb]

        idxg, tblA = pl.pallas_call(
            _topk_cell,
            grid=(nb, _NBLK),
            in_specs=[
                pl.BlockSpec((1, 3, _N), lambda b, m: (b, 0, 0)),
                pl.BlockSpec((1, _N, 3), lambda b, m: (b, 0, 0)),
            ] + wspecsA,
            out_specs=[
                pl.BlockSpec((1, _M, _K), lambda b, m: (b, m, 0)),
                pl.BlockSpec((1, _M, _D), lambda b, m: (b, m, 0)),
            ],
            out_shape=[
                jax.ShapeDtypeStruct((nb, _N, _K), jnp.int32),
                jax.ShapeDtypeStruct((nb, _N, _D), f32),
            ],
            scratch_shapes=[pltpu.VMEM((_M, _N), jnp.int32)],
        )(xyz_g, xyzT_g, *wsA)

        tbl = tblA.reshape(nb * _N, _D)
        idx_flat = jnp.transpose(idxg, (2, 0, 1)).reshape(tot)

        G = pl.kernel(
            sc_gather,
            mesh=mesh,
            out_type=jax.ShapeDtypeStruct((tot, _D), f32),
            scratch_types=[
                pltpu.VMEM((pw,), jnp.int32),
                pltpu.VMEM((_CH, _D), f32),
                pltpu.VMEM((_CH, _D), f32),
                pltpu.SemaphoreType.DMA,
                pltpu.SemaphoreType.DMA,
            ],
        )(tbl, idx_flat)

        G4 = G.reshape(_K, nb, _N, _D)
        out_pm = pl.pallas_call(
            _mlp_cell,
            grid=(nb, _NBLK),
            in_specs=[
                pl.BlockSpec((_K, 1, _M, _D), lambda b, m: (0, b, m, 0)),
            ] + wspecsC,
            out_specs=pl.BlockSpec((1, _M, 6), lambda b, m: (b, m, 0)),
            out_shape=jax.ShapeDtypeStruct((nb, _N, 6), f32),
        )(G4, *wsC)
        outs.append(out_pm)

    out_all = jnp.concatenate(outs, axis=0)
    return out_all.transpose(0, 2, 1).reshape(Bsz, 3, _R * Np)


# speedup vs baseline: 1.0330x; 1.0001x over previous
import functools

import jax
import jax.numpy as jnp
from jax import lax
from jax.experimental import pallas as pl
from jax.experimental.pallas import tpu as pltpu
from jax.experimental.pallas import tpu_sc as plsc

_B, _N, _K, _R = 8, 2048, 16, 2
_M = 512
_NBLK = _N // _M
_D = 128
_NW = 32
_TOT = _B * _N * _K
_PW = _TOT // _NW
_CH = 128
_NCH = _PW // _CH


def _topk_cell(xyz_ref, xyzT_ref,
               W1T_ref, b1_ref, W2T_ref, b2_ref,
               idx_ref, tbl_ref, kk_ref):
    b = pl.program_id(0)
    m = pl.program_id(1)
    f32 = jnp.float32
    x3 = xyz_ref[0]
    Q = xyzT_ref[0, pl.ds(m * _M, _M), :]

    P1 = jnp.maximum(jnp.dot(Q, W1T_ref[...], preferred_element_type=f32)
                     + b1_ref[...], 0.0)
    fq = jnp.maximum(jnp.dot(P1, W2T_ref[...], preferred_element_type=f32)
                     + b2_ref[...], 0.0)
    tbl_ref[0] = jnp.concatenate(
        [fq, Q, jnp.zeros((_M, _D - 67), f32)], axis=1)

    sq_all = jnp.sum(x3 * x3, axis=0, keepdims=True)
    sq_q = jnp.sum(Q * Q, axis=1, keepdims=True)
    dotQ = jnp.dot(Q, x3, preferred_element_type=f32)
    lane = lax.broadcasted_iota(jnp.int32, (_M, 128), 1)
    rid = lax.broadcasted_iota(jnp.int32, (_M, 1), 0) + m * _M
    nch = _N // 128

    base = b * _N
    self_col = rid + base
    maxi = jnp.int32(0x7FFFFFFF)

    def glob2(a1, a2):
        m1 = jnp.min(a1, axis=1, keepdims=True)
        m2 = jnp.min(jnp.where(a1 == m1, a2, a1), axis=1, keepdims=True)
        return m1, m2

    a1 = a2 = None
    for j in range(nch):
        sl = pl.ds(j * 128, 128)
        lo, hi = j * 128, (j + 1) * 128
        d2c = jnp.maximum(sq_q + sq_all[:, lo:hi] - 2.0 * dotQ[:, lo:hi], 0.0)
        lj = lane + (j * 128)
        bch = (lax.bitcast_convert_type(d2c, jnp.int32) & jnp.int32(-2048)) | lj
        bch = jnp.where(lj == rid, maxi, bch)
        kk_ref[:, sl] = bch
        if a1 is None:
            a1, a2 = bch, jnp.full_like(bch, maxi)
        else:
            t = jnp.maximum(a1, bch)
            a1 = jnp.minimum(a1, bch)
            a2 = jnp.minimum(a2, t)
    m1, m2 = glob2(a1, a2)
    cols = [self_col, (m1 & 2047) + base, (m2 & 2047) + base]
    for _ in range(6):
        prev2 = m2
        a1 = a2 = None
        for j in range(nch):
            sl = pl.ds(j * 128, 128)
            c = kk_ref[:, sl]
            c = jnp.where(c <= prev2, maxi, c)
            if a1 is None:
                a1, a2 = c, jnp.full_like(c, maxi)
            else:
                t = jnp.maximum(a1, c)
                a1 = jnp.minimum(a1, c)
                a2 = jnp.minimum(a2, t)
        m1, m2 = glob2(a1, a2)
        cols.append((m1 & 2047) + base)
        cols.append((m2 & 2047) + base)
    prev2 = m2
    macc = None
    for j in range(nch):
        lo, hi = j * 128, (j + 1) * 128
        c = kk_ref[:, pl.ds(j * 128, 128)]
        c = jnp.where(c <= prev2, maxi, c)
        macc = c if macc is None else jnp.minimum(macc, c)
    mlast = jnp.min(macc, axis=1, keepdims=True)
    cols.append((mlast & 2047) + base)
    idx_ref[0] = jnp.concatenate(cols, axis=1)


def _make_sc_gather(pw, nch):
  def _sc_gather(tbl_hbm, idx_hbm, out_hbm, idx_v, buf0, buf1, sem0, sem1):
    wid = lax.axis_index("s") * 2 + lax.axis_index("c")
    base = wid * pw
    pltpu.sync_copy(idx_hbm.at[pl.ds(base, pw)], idx_v)

    def start(st, buf, sem):
        pltpu.async_copy(tbl_hbm.at[idx_v.at[pl.ds(st, _CH)]], buf, sem)

    def wait(buf, sem):
        pltpu.make_async_copy(tbl_hbm.at[pl.ds(0, _CH)], buf, sem).wait()

    def out(st, buf):
        pltpu.sync_copy(buf, out_hbm.at[pl.ds(base + st, _CH)])

    start(0, buf0, sem0)

    def body(i, carry):
        st = 2 * i * _CH
        start(st + _CH, buf1, sem1)
        wait(buf0, sem0)
        out(st, buf0)
        start(st + 2 * _CH, buf0, sem0)
        wait(buf1, sem1)
        out(st + _CH, buf1)
        return carry

    lax.fori_loop(0, nch // 2 - 1, body, 0)
    st = (nch - 2) * _CH
    start(st + _CH, buf1, sem1)
    wait(buf0, sem0)
    out(st, buf0)
    wait(buf1, sem1)
    out(st + _CH, buf1)
  return _sc_gather


def _mlp_cell(g_ref,
              WpT_ref, WsT_ref, bs_ref,
              Wa1T_ref, ba1_ref, Wa2T_ref, ba2_ref,
              Wm1aT_ref, Wm1bT_ref, bm1_ref, Wm2T_ref, bm2_ref,
              out_ref):
    f32 = jnp.float32
    Gb = g_ref[...]
    nb0 = Gb[0, 0]
    fq = nb0[:, 0:64]
    Q = nb0[:, 64:67]

    WpT = WpT_ref[...]
    Wa1T = Wa1T_ref[...]
    ba1 = ba1_ref[...]
    Wa2T = Wa2T_ref[...]
    ba2 = ba2_ref[...]

    lg0 = jnp.dot(jnp.maximum(ba1, 0.0), Wa2T, preferred_element_type=f32) + ba2

    logits = [lg0]
    fks = []
    planes = None
    for k in range(1, _K):
        nb = Gb[k, 0]
        nbf = nb[:, 0:64]
        nbx = nb[:, 64:67]

        fk = nbf - fq
        a = jnp.maximum(jnp.dot(fk, Wa1T, preferred_element_type=f32) + ba1, 0.0)
        lg = jnp.dot(a, Wa2T, preferred_element_type=f32) + ba2
        logits.append(lg)
        fks.append(fk)

        v = nbx - Q
        nrm = jnp.sqrt(jnp.sum(v * v, axis=1, keepdims=True)) + 1e-8
        p = jnp.dot(v, WpT, preferred_element_type=f32)
        c = p * jnp.abs(p) / nrm
        planes = c if planes is None else jnp.maximum(planes, c)

    mx = functools.reduce(jnp.maximum, logits)
    den = jnp.exp(jnp.broadcast_to(logits[0], mx.shape) - mx)
    num = None
    for k in range(1, _K):
        e = jnp.exp(logits[k] - mx)
        den = den + e
        t = e * fks[k - 1]
        num = t if num is None else num + t
    f_att = num / den

    f_shapes = jnp.dot(planes, WsT_ref[...], preferred_element_type=f32) + bs_ref[...]

    h = jnp.maximum(jnp.dot(f_att, Wm1aT_ref[...], preferred_element_type=f32)
                    + jnp.dot(f_shapes, Wm1bT_ref[...], preferred_element_type=f32)
                    + bm1_ref[...], 0.0)
    o = jnp.dot(h, Wm2T_ref[...], preferred_element_type=f32) + bm2_ref[...]

    Qe = jnp.concatenate([Q[:, 0:1], Q[:, 0:1], Q[:, 1:2], Q[:, 1:2],
                          Q[:, 2:3], Q[:, 2:3]], axis=1)
    out_ref[0] = Qe + 0.15 * o


def kernel(xyz, Wp, Ws, bs, W1, b1, W2, b2, Wa1, ba1, Wa2, ba2, Wm1, bm1, Wm2, bm2):
    Bsz, C, Np = xyz.shape
    f32 = jnp.float32
    xyzT = jnp.transpose(xyz, (0, 2, 1))

    row = lambda v: v.reshape(1, -1)
    wsA = [W1.T, row(b1), W2.T, row(b2)]
    wspecsA = [pl.BlockSpec(w.shape, lambda b, m: (0,) * w.ndim) for w in wsA]
    wsC = [Wp.T, Ws.T, row(bs), Wa1.T, row(ba1), Wa2.T, row(ba2),
           Wm1[:, :64].T, Wm1[:, 64:].T, row(bm1), Wm2.T, row(bm2)]
    wspecsC = [pl.BlockSpec(w.shape, lambda b, m: (0,) * w.ndim) for w in wsC]
    mesh = plsc.VectorSubcoreMesh(core_axis_name="c", subcore_axis_name="s")

    _NG = 2
    nb = _B // _NG
    tot = nb * _N * _K
    pw = tot // _NW
    nch = pw // _CH
    sc_gather = _make_sc_gather(pw, nch)

    outs = []
    for g in range(_NG):
        xyz_g = xyz[g * nb:(g + 1) * nb]
        xyzT_g = xyzT[g * nb:(g + 1) * nb]

        idxg, tblA = pl.pallas_call(
            _topk_cell,
            grid=(nb, _NBLK),
            in_specs=[
                pl.BlockSpec((1, 3, _N), lambda b, m: (b, 0, 0)),
                pl.BlockSpec((1, _N, 3), lambda b, m: (b, 0, 0)),
            ] + wspecsA,
            out_specs=[
                pl.BlockSpec((1, _M, _K), lambda b, m: (b, m, 0)),
                pl.BlockSpec((1, _M, _D), lambda b, m: (b, m, 0)),
            ],
            out_shape=[
                jax.ShapeDtypeStruct((nb, _N, _K), jnp.int32),
                jax.ShapeDtypeStruct((nb, _N, _D), f32),
            ],
            scratch_shapes=[pltpu.VMEM((_M, _N), jnp.int32)],
        )(xyz_g, xyzT_g, *wsA)

        tbl = tblA.reshape(nb * _N, _D)
        idx_flat = jnp.transpose(idxg, (2, 0, 1)).reshape(tot)

        G = pl.kernel(
            sc_gather,
            mesh=mesh,
            out_type=jax.ShapeDtypeStruct((tot, _D), f32),
            scratch_types=[
                pltpu.VMEM((pw,), jnp.int32),
                pltpu.VMEM((_CH, _D), f32),
                pltpu.VMEM((_CH, _D), f32),
                pltpu.SemaphoreType.DMA,
                pltpu.SemaphoreType.DMA,
            ],
        )(tbl, idx_flat)

        G4 = G.reshape(_K, nb, _N, _D)
        out_pm = pl.pallas_call(
            _mlp_cell,
            grid=(nb, _NBLK),
            in_specs=[
                pl.BlockSpec((_K, 1, _M, _D), lambda b, m: (0, b, m, 0)),
            ] + wspecsC,
            out_specs=pl.BlockSpec((1, _M, 6), lambda b, m: (b, m, 0)),
            out_shape=jax.ShapeDtypeStruct((nb, _N, 6), f32),
        )(G4, *wsC)
        outs.append(out_pm)

    out_all = jnp.concatenate(outs, axis=0)
    return out_all.transpose(0, 2, 1).reshape(Bsz, 3, _R * Np)


# speedup vs baseline: 1.3107x; 1.2688x over previous
import functools

import jax
import jax.numpy as jnp
from jax import lax
from jax.experimental import pallas as pl
from jax.experimental.pallas import tpu as pltpu
from jax.experimental.pallas import tpu_sc as plsc

_B, _N, _K, _R = 8, 2048, 16, 2
_M = 512
_NBLK = _N // _M
_D = 128
_NW = 32
_TOT = _B * _N * _K
_PW = _TOT // _NW
_CH = 128
_NCH = _PW // _CH


def _topk_cell(xyz_ref, xyzT_ref,
               W1T_ref, b1_ref, W2T_ref, b2_ref,
               idx_ref, tbl_ref, kk_ref):
    b = pl.program_id(0)
    m = pl.program_id(1)
    f32 = jnp.float32
    x3 = xyz_ref[0]
    Q = xyzT_ref[0, pl.ds(m * _M, _M), :]

    P1 = jnp.maximum(jnp.dot(Q, W1T_ref[...], preferred_element_type=f32)
                     + b1_ref[...], 0.0)
    fq = jnp.maximum(jnp.dot(P1, W2T_ref[...], preferred_element_type=f32)
                     + b2_ref[...], 0.0)
    tbl_ref[0] = jnp.concatenate(
        [fq, Q, jnp.zeros((_M, _D - 67), f32)], axis=1)

    sq_all = jnp.sum(x3 * x3, axis=0, keepdims=True)
    sq_q = jnp.sum(Q * Q, axis=1, keepdims=True)
    dotQ = jnp.dot(Q, x3, preferred_element_type=f32)
    lane = lax.broadcasted_iota(jnp.int32, (_M, 128), 1)
    rid = lax.broadcasted_iota(jnp.int32, (_M, 1), 0) + m * _M
    nch = _N // 128

    base = b * _N
    self_col = rid + base
    inf = jnp.float32(jnp.inf)

    def exid(mv):
        return (lax.bitcast_convert_type(mv, jnp.int32) & 2047) + base

    def glob2(a1, a2):
        m1 = jnp.min(a1, axis=1, keepdims=True)
        m2 = jnp.min(jnp.where(a1 == m1, a2, a1), axis=1, keepdims=True)
        return m1, m2

    a1 = a2 = None
    for j in range(nch):
        sl = pl.ds(j * 128, 128)
        lo, hi = j * 128, (j + 1) * 128
        d2c = sq_q + sq_all[:, lo:hi] - 2.0 * dotQ[:, lo:hi]
        d2c = jnp.maximum(d2c, 0.0) + 1.0
        lj = lane + (j * 128)
        bch = lax.bitcast_convert_type(
            (lax.bitcast_convert_type(d2c, jnp.int32) & jnp.int32(-2048)) | lj,
            jnp.float32)
        bch = jnp.where(lj == rid, inf, bch)
        kk_ref[:, sl] = bch
        if a1 is None:
            a1, a2 = bch, jnp.full_like(bch, inf)
        else:
            t = jnp.maximum(a1, bch)
            a1 = jnp.minimum(a1, bch)
            a2 = jnp.minimum(a2, t)
    m1, m2 = glob2(a1, a2)
    cols = [self_col, exid(m1), exid(m2)]
    for _ in range(6):
        prev2 = m2
        a1 = a2 = None
        for j in range(nch):
            sl = pl.ds(j * 128, 128)
            c = kk_ref[:, sl]
            c = jnp.where(c <= prev2, inf, c)
            if a1 is None:
                a1, a2 = c, jnp.full_like(c, inf)
            else:
                t = jnp.maximum(a1, c)
                a1 = jnp.minimum(a1, c)
                a2 = jnp.minimum(a2, t)
        m1, m2 = glob2(a1, a2)
        cols.append(exid(m1))
        cols.append(exid(m2))
    prev2 = m2
    macc = None
    for j in range(nch):
        c = kk_ref[:, pl.ds(j * 128, 128)]
        c = jnp.where(c <= prev2, inf, c)
        macc = c if macc is None else jnp.minimum(macc, c)
    mlast = jnp.min(macc, axis=1, keepdims=True)
    cols.append(exid(mlast))
    idx_ref[0] = jnp.concatenate(cols, axis=1)


def _make_sc_gather(pw, nch):
  def _sc_gather(tbl_hbm, idx_hbm, out_hbm, idx_v, buf0, buf1, sem0, sem1):
    wid = lax.axis_index("s") * 2 + lax.axis_index("c")
    base = wid * pw
    pltpu.sync_copy(idx_hbm.at[pl.ds(base, pw)], idx_v)

    def start(st, buf, sem):
        pltpu.async_copy(tbl_hbm.at[idx_v.at[pl.ds(st, _CH)]], buf, sem)

    def wait(buf, sem):
        pltpu.make_async_copy(tbl_hbm.at[pl.ds(0, _CH)], buf, sem).wait()

    def out(st, buf):
        pltpu.sync_copy(buf, out_hbm.at[pl.ds(base + st, _CH)])

    start(0, buf0, sem0)

    def body(i, carry):
        st = 2 * i * _CH
        start(st + _CH, buf1, sem1)
        wait(buf0, sem0)
        out(st, buf0)
        start(st + 2 * _CH, buf0, sem0)
        wait(buf1, sem1)
        out(st + _CH, buf1)
        return carry

    lax.fori_loop(0, nch // 2 - 1, body, 0)
    st = (nch - 2) * _CH
    start(st + _CH, buf1, sem1)
    wait(buf0, sem0)
    out(st, buf0)
    wait(buf1, sem1)
    out(st + _CH, buf1)
  return _sc_gather


def _mlp_cell(g_ref,
              WpT_ref, WsT_ref, bs_ref,
              Wa1T_ref, ba1_ref, Wa2T_ref, ba2_ref,
              Wm1aT_ref, Wm1bT_ref, bm1_ref, Wm2T_ref, bm2_ref,
              out_ref):
    f32 = jnp.float32
    Gb = g_ref[...]
    nb0 = Gb[0, 0]
    fq = nb0[:, 0:64]
    Q = nb0[:, 64:67]

    WpT = WpT_ref[...]
    Wa1T = Wa1T_ref[...]
    ba1 = ba1_ref[...]
    Wa2T = Wa2T_ref[...]
    ba2 = ba2_ref[...]

    lg0 = jnp.dot(jnp.maximum(ba1, 0.0), Wa2T, preferred_element_type=f32) + ba2

    logits = [lg0]
    fks = []
    planes = None
    for k in range(1, _K):
        nb = Gb[k, 0]
        nbf = nb[:, 0:64]
        nbx = nb[:, 64:67]

        fk = nbf - fq
        a = jnp.maximum(jnp.dot(fk, Wa1T, preferred_element_type=f32) + ba1, 0.0)
        lg = jnp.dot(a, Wa2T, preferred_element_type=f32) + ba2
        logits.append(lg)
        fks.append(fk)

        v = nbx - Q
        nrm = jnp.sqrt(jnp.sum(v * v, axis=1, keepdims=True)) + 1e-8
        p = jnp.dot(v, WpT, preferred_element_type=f32)
        c = p * jnp.abs(p) / nrm
        planes = c if planes is None else jnp.maximum(planes, c)

    mx = functools.reduce(jnp.maximum, logits)
    den = jnp.exp(jnp.broadcast_to(logits[0], mx.shape) - mx)
    num = None
    for k in range(1, _K):
        e = jnp.exp(logits[k] - mx)
        den = den + e
        t = e * fks[k - 1]
        num = t if num is None else num + t
    f_att = num / den

    f_shapes = jnp.dot(planes, WsT_ref[...], preferred_element_type=f32) + bs_ref[...]

    h = jnp.maximum(jnp.dot(f_att, Wm1aT_ref[...], preferred_element_type=f32)
                    + jnp.dot(f_shapes, Wm1bT_ref[...], preferred_element_type=f32)
                    + bm1_ref[...], 0.0)
    o = jnp.dot(h, Wm2T_ref[...], preferred_element_type=f32) + bm2_ref[...]

    Qe = jnp.concatenate([Q[:, 0:1], Q[:, 0:1], Q[:, 1:2], Q[:, 1:2],
                          Q[:, 2:3], Q[:, 2:3]], axis=1)
    out_ref[0] = Qe + 0.15 * o


def kernel(xyz, Wp, Ws, bs, W1, b1, W2, b2, Wa1, ba1, Wa2, ba2, Wm1, bm1, Wm2, bm2):
    Bsz, C, Np = xyz.shape
    f32 = jnp.float32
    xyzT = jnp.transpose(xyz, (0, 2, 1))

    row = lambda v: v.reshape(1, -1)
    wsA = [W1.T, row(b1), W2.T, row(b2)]
    wspecsA = [pl.BlockSpec(w.shape, lambda b, m: (0,) * w.ndim) for w in wsA]
    wsC = [Wp.T, Ws.T, row(bs), Wa1.T, row(ba1), Wa2.T, row(ba2),
           Wm1[:, :64].T, Wm1[:, 64:].T, row(bm1), Wm2.T, row(bm2)]
    wspecsC = [pl.BlockSpec(w.shape, lambda b, m: (0,) * w.ndim) for w in wsC]
    mesh = plsc.VectorSubcoreMesh(core_axis_name="c", subcore_axis_name="s")

    _NG = 2
    nb = _B // _NG
    tot = nb * _N * _K
    pw = tot // _NW
    nch = pw // _CH
    sc_gather = _make_sc_gather(pw, nch)

    outs = []
    for g in range(_NG):
        xyz_g = xyz[g * nb:(g + 1) * nb]
        xyzT_g = xyzT[g * nb:(g + 1) * nb]

        idxg, tblA = pl.pallas_call(
            _topk_cell,
            grid=(nb, _NBLK),
            in_specs=[
                pl.BlockSpec((1, 3, _N), lambda b, m: (b, 0, 0)),
                pl.BlockSpec((1, _N, 3), lambda b, m: (b, 0, 0)),
            ] + wspecsA,
            out_specs=[
                pl.BlockSpec((1, _M, _K), lambda b, m: (b, m, 0)),
                pl.BlockSpec((1, _M, _D), lambda b, m: (b, m, 0)),
            ],
            out_shape=[
                jax.ShapeDtypeStruct((nb, _N, _K), jnp.int32),
                jax.ShapeDtypeStruct((nb, _N, _D), f32),
            ],
            scratch_shapes=[pltpu.VMEM((_M, _N), jnp.float32)],
        )(xyz_g, xyzT_g, *wsA)

        tbl = tblA.reshape(nb * _N, _D)
        idx_flat = jnp.transpose(idxg, (2, 0, 1)).reshape(tot)

        G = pl.kernel(
            sc_gather,
            mesh=mesh,
            out_type=jax.ShapeDtypeStruct((tot, _D), f32),
            scratch_types=[
                pltpu.VMEM((pw,), jnp.int32),
                pltpu.VMEM((_CH, _D), f32),
                pltpu.VMEM((_CH, _D), f32),
                pltpu.SemaphoreType.DMA,
                pltpu.SemaphoreType.DMA,
            ],
        )(tbl, idx_flat)

        G4 = G.reshape(_K, nb, _N, _D)
        out_pm = pl.pallas_call(
            _mlp_cell,
            grid=(nb, _NBLK),
            in_specs=[
                pl.BlockSpec((_K, 1, _M, _D), lambda b, m: (0, b, m, 0)),
            ] + wspecsC,
            out_specs=pl.BlockSpec((1, _M, 6), lambda b, m: (b, m, 0)),
            out_shape=jax.ShapeDtypeStruct((nb, _N, 6), f32),
        )(G4, *wsC)
        outs.append(out_pm)

    out_all = jnp.concatenate(outs, axis=0)
    return out_all.transpose(0, 2, 1).reshape(Bsz, 3, _R * Np)
